# baseline (speedup 1.0000x reference)
import math

import jax
import jax.numpy as jnp
from jax import lax
from jax.experimental import pallas as pl
from jax.experimental.pallas import tpu as pltpu

_N = 5000
_NP = 5120
_FEAT = 1024
_NC = 80
_SCORE_THRESH = 0.05
_NMS_THRESH = 0.5
_DET = 100
_SCALE_CLAMP = math.log(1000.0 / 16.0)
_ROWS = 512
_GRID = _NP // _ROWS


def _dense_body(x_ref, bxt_ref, w_ref, b_ref, s_ref, x0_ref, y0_ref, x1_ref, y1_ref):
    xb = x_ref[...]
    acc = lax.dot_general(w_ref[...], xb, (((0,), (1,)), ((), ())),
                          preferred_element_type=jnp.float32)
    acc = acc + b_ref[...]
    ci = lax.broadcasted_iota(jnp.int32, acc.shape, 0)
    neg = -jnp.inf
    m_all = jnp.max(jnp.where(ci < _NC + 1, acc, neg), axis=0, keepdims=True)
    m_fg = jnp.max(jnp.where(ci < _NC, acc, neg), axis=0, keepdims=True)
    e = jnp.exp(jnp.where(ci < _NC + 1, acc - m_all, neg))
    s_sum = jnp.sum(e, axis=0, keepdims=True)
    score = jnp.exp(m_fg - m_all) / s_sum

    dx = acc[81:82, :] / 10.0
    dy = acc[82:83, :] / 10.0
    dw = jnp.minimum(acc[83:84, :] / 5.0, _SCALE_CLAMP)
    dh = jnp.minimum(acc[84:85, :] / 5.0, _SCALE_CLAMP)
    px0 = bxt_ref[0:1, :]
    py0 = bxt_ref[1:2, :]
    px1 = bxt_ref[2:3, :]
    py1 = bxt_ref[3:4, :]
    widths = px1 - px0
    heights = py1 - py0
    ctr_x = px0 + 0.5 * widths
    ctr_y = py0 + 0.5 * heights
    pcx = dx * widths + ctr_x
    pcy = dy * heights + ctr_y
    pw = jnp.exp(dw) * widths
    ph = jnp.exp(dh) * heights
    s_ref[...] = score
    x0_ref[...] = pcx - 0.5 * pw
    y0_ref[...] = pcy - 0.5 * ph
    x1_ref[...] = pcx + 0.5 * pw
    y1_ref[...] = pcy + 0.5 * ph


def _allmax(a):
    return jnp.max(a, axis=(0, 1), keepdims=True)


def _select_body(s_ref, x0_ref, y0_ref, x1_ref, y1_ref, o_ref,
                 a_ref, b_ref, area_ref):
    sc = s_ref[...]
    bx0 = x0_ref[...]
    by0 = y0_ref[...]
    bx1 = x1_ref[...]
    by1 = y1_ref[...]
    fr = lax.broadcasted_iota(jnp.int32, sc.shape, 0)
    fc = lax.broadcasted_iota(jnp.int32, sc.shape, 1)
    flatf = (fr * 128 + fc).astype(jnp.float32)
    real = (fr * 128 + fc) < _N
    valid = real & (sc > _SCORE_THRESH)
    neg = -jnp.inf
    a_ref[...] = jnp.where(valid, sc, neg)
    b_ref[...] = jnp.where(real, jnp.where(valid, sc, -1.0), neg)
    area_ref[...] = (bx1 - bx0) * (by1 - by0)
    o_ref[...] = jnp.zeros(o_ref.shape, jnp.float32)

    def body(t, carry):
        a = a_ref[...]
        b = b_ref[...]
        m1 = _allmax(a)
        m2 = _allmax(b)
        p1 = m1 > jnp.float32(-1e30)
        pool = jnp.where(p1, a, b)
        m = jnp.where(p1, m1, m2)
        cand = pool == m

        def pickm(msk, arr):
            return jnp.max(jnp.where(msk, arr, neg), axis=(0, 1), keepdims=True)

        candf = jnp.where(cand, 1.0, 0.0)
        nc = jnp.sum(candf)
        sx0 = pickm(cand, bx0)
        sy0 = pickm(cand, by0)
        sx1 = pickm(cand, bx1)
        sy1 = pickm(cand, by1)
        ssc = pickm(cand, sc)

        def fast():
            return sx0, sy0, sx1, sy1, ssc, candf

        def slow():
            c2 = candf > 0.5
            j = jnp.min(jnp.where(c2, flatf, jnp.float32(jnp.inf)),
                        axis=(0, 1), keepdims=True)
            s2 = flatf == j
            return (pickm(s2, bx0), pickm(s2, by0), pickm(s2, bx1),
                    pickm(s2, by1), pickm(s2, sc), jnp.where(s2, 1.0, 0.0))

        jx0, jy0, jx1, jy1, jsc, self_f = lax.cond(nc == 1.0, fast, slow)
        sel = self_f > 0.5
        jar = (jx1 - jx0) * (jy1 - jy0)
        w = jnp.maximum(jnp.minimum(bx1, jx1) - jnp.maximum(bx0, jx0), 0.0)
        h = jnp.maximum(jnp.minimum(by1, jy1) - jnp.maximum(by0, jy0), 0.0)
        inter = w * h
        iou = inter / (jar + area_ref[...] - inter + 1e-9)
        supp = jnp.logical_and(p1, iou > _NMS_THRESH)
        a_ref[...] = jnp.where(supp | sel, neg, a)
        b_ref[...] = jnp.where(sel, neg, b)

        sub8 = lax.broadcasted_iota(jnp.int32, (8, 128), 0)
        lane8 = lax.broadcasted_iota(jnp.int32, (8, 128), 1)
        vals = jnp.where(sub8 == 0, jx0,
               jnp.where(sub8 == 1, jy0,
               jnp.where(sub8 == 2, jx1,
               jnp.where(sub8 == 3, jy1, jsc))))
        o_ref[...] = o_ref[...] + jnp.where(lane8 == t, vals, 0.0)
        return carry

    lax.fori_loop(0, _DET, body, 0)


def kernel(box_features, proposal_boxes, W_cls, b_cls, W_box, b_box):
    f32 = jnp.float32
    w_all = jnp.zeros((_FEAT, 128), f32)
    w_all = w_all.at[:, : _NC + 1].set(W_cls).at[:, _NC + 1 : _NC + 5].set(W_box)
    b_all = jnp.zeros((128, 1), f32)
    b_all = b_all.at[: _NC + 1, 0].set(b_cls).at[_NC + 1 : _NC + 5, 0].set(b_box)
    boxes_t = proposal_boxes.T

    row = jax.ShapeDtypeStruct((1, _NP), f32)
    score, x0, y0, x1, y1 = pl.pallas_call(
        _dense_body,
        grid=(_GRID,),
        in_specs=[
            pl.BlockSpec((_ROWS, _FEAT), lambda i: (i, 0)),
            pl.BlockSpec((4, _ROWS), lambda i: (0, i)),
            pl.BlockSpec((_FEAT, 128), lambda i: (0, 0)),
            pl.BlockSpec((128, 1), lambda i: (0, 0)),
        ],
        out_specs=[pl.BlockSpec((1, _ROWS), lambda i: (0, i))] * 5,
        out_shape=[row] * 5,
    )(box_features, boxes_t, w_all, b_all)

    lane = lambda a: a.reshape(_NP // 128, 128)
    out8 = pl.pallas_call(
        _select_body,
        out_shape=jax.ShapeDtypeStruct((8, 128), f32),
        scratch_shapes=[pltpu.VMEM((_NP // 128, 128), f32)] * 3,
    )(lane(score), lane(x0), lane(y0), lane(x1), lane(y1))
    return out8[:5, :_DET].T


# speedup vs baseline: 1.4176x; 1.4176x over previous
import math

import jax
import jax.numpy as jnp
from jax import lax
from jax.experimental import pallas as pl
from jax.experimental.pallas import tpu as pltpu

_N = 5000
_NP = 5120
_FEAT = 1024
_NC = 80
_SCORE_THRESH = 0.05
_NMS_THRESH = 0.5
_DET = 100
_SCALE_CLAMP = math.log(1000.0 / 16.0)
_ROWS = 512
_GRID = _NP // _ROWS


def _dense_body(x_ref, bxt_ref, w_ref, b_ref, s_ref, x0_ref, y0_ref, x1_ref, y1_ref):
    xb = x_ref[...]
    acc = lax.dot_general(w_ref[...], xb, (((0,), (1,)), ((), ())),
                          preferred_element_type=jnp.float32)
    acc = acc + b_ref[...]
    ci = lax.broadcasted_iota(jnp.int32, acc.shape, 0)
    neg = -jnp.inf
    m_all = jnp.max(jnp.where(ci < _NC + 1, acc, neg), axis=0, keepdims=True)
    m_fg = jnp.max(jnp.where(ci < _NC, acc, neg), axis=0, keepdims=True)
    e = jnp.exp(jnp.where(ci < _NC + 1, acc - m_all, neg))
    s_sum = jnp.sum(e, axis=0, keepdims=True)
    score = jnp.exp(m_fg - m_all) / s_sum

    dx = acc[81:82, :] / 10.0
    dy = acc[82:83, :] / 10.0
    dw = jnp.minimum(acc[83:84, :] / 5.0, _SCALE_CLAMP)
    dh = jnp.minimum(acc[84:85, :] / 5.0, _SCALE_CLAMP)
    px0 = bxt_ref[0:1, :]
    py0 = bxt_ref[1:2, :]
    px1 = bxt_ref[2:3, :]
    py1 = bxt_ref[3:4, :]
    widths = px1 - px0
    heights = py1 - py0
    ctr_x = px0 + 0.5 * widths
    ctr_y = py0 + 0.5 * heights
    pcx = dx * widths + ctr_x
    pcy = dy * heights + ctr_y
    pw = jnp.exp(dw) * widths
    ph = jnp.exp(dh) * heights
    s_ref[...] = score
    x0_ref[...] = pcx - 0.5 * pw
    y0_ref[...] = pcy - 0.5 * ph
    x1_ref[...] = pcx + 0.5 * pw
    y1_ref[...] = pcy + 0.5 * ph


def _allmax(a):
    return jnp.max(jnp.max(a, axis=0, keepdims=True), axis=1, keepdims=True)


def _select_body(s_ref, x0_ref, y0_ref, x1_ref, y1_ref, o_ref,
                 a_ref, b_ref, area_ref):
    sc = s_ref[...]
    bx0 = x0_ref[...]
    by0 = y0_ref[...]
    bx1 = x1_ref[...]
    by1 = y1_ref[...]
    fr = lax.broadcasted_iota(jnp.int32, sc.shape, 0)
    fc = lax.broadcasted_iota(jnp.int32, sc.shape, 1)
    flatf = (fr * 128 + fc).astype(jnp.float32)
    real = (fr * 128 + fc) < _N
    valid = real & (sc > _SCORE_THRESH)
    neg = -jnp.inf
    a_ref[...] = jnp.where(valid, sc, neg)
    b_ref[...] = jnp.where(real, jnp.where(valid, sc, -1.0), neg)
    area_ref[...] = (bx1 - bx0) * (by1 - by0)
    o_ref[...] = jnp.zeros(o_ref.shape, jnp.float32)

    def body(t, carry):
        a = a_ref[...]
        b = b_ref[...]
        m1 = _allmax(a)
        m2 = _allmax(b)
        p1 = m1 > jnp.float32(-1e30)
        pool = jnp.where(p1, a, b)
        m = jnp.where(p1, m1, m2)
        cand = pool == m

        def pickm(msk, arr):
            return jnp.max(jnp.max(jnp.where(msk, arr, neg),
                                   axis=0, keepdims=True), axis=1, keepdims=True)

        j = jnp.min(jnp.min(jnp.where(cand, flatf, jnp.float32(jnp.inf)),
                            axis=0, keepdims=True), axis=1, keepdims=True)
        sel = flatf == j

        jx0 = pickm(sel, bx0)
        jy0 = pickm(sel, by0)
        jx1 = pickm(sel, bx1)
        jy1 = pickm(sel, by1)
        jsc = pickm(sel, sc)
        jar = (jx1 - jx0) * (jy1 - jy0)
        w = jnp.maximum(jnp.minimum(bx1, jx1) - jnp.maximum(bx0, jx0), 0.0)
        h = jnp.maximum(jnp.minimum(by1, jy1) - jnp.maximum(by0, jy0), 0.0)
        inter = w * h
        iou = inter / (jar + area_ref[...] - inter + 1e-9)
        supp = jnp.logical_and(p1, iou > _NMS_THRESH)
        a_ref[...] = jnp.where(supp | sel, neg, a)
        b_ref[...] = jnp.where(sel, neg, b)

        sub8 = lax.broadcasted_iota(jnp.int32, (8, 128), 0)
        lane8 = lax.broadcasted_iota(jnp.int32, (8, 128), 1)
        vals = jnp.where(sub8 == 0, jx0,
               jnp.where(sub8 == 1, jy0,
               jnp.where(sub8 == 2, jx1,
               jnp.where(sub8 == 3, jy1, jsc))))
        o_ref[...] = o_ref[...] + jnp.where(lane8 == t, vals, 0.0)
        return carry

    lax.fori_loop(0, _DET, body, 0)


def kernel(box_features, proposal_boxes, W_cls, b_cls, W_box, b_box):
    f32 = jnp.float32
    w_all = jnp.zeros((_FEAT, 128), f32)
    w_all = w_all.at[:, : _NC + 1].set(W_cls).at[:, _NC + 1 : _NC + 5].set(W_box)
    b_all = jnp.zeros((128, 1), f32)
    b_all = b_all.at[: _NC + 1, 0].set(b_cls).at[_NC + 1 : _NC + 5, 0].set(b_box)
    boxes_t = proposal_boxes.T

    row = jax.ShapeDtypeStruct((1, _NP), f32)
    score, x0, y0, x1, y1 = pl.pallas_call(
        _dense_body,
        grid=(_GRID,),
        in_specs=[
            pl.BlockSpec((_ROWS, _FEAT), lambda i: (i, 0)),
            pl.BlockSpec((4, _ROWS), lambda i: (0, i)),
            pl.BlockSpec((_FEAT, 128), lambda i: (0, 0)),
            pl.BlockSpec((128, 1), lambda i: (0, 0)),
        ],
        out_specs=[pl.BlockSpec((1, _ROWS), lambda i: (0, i))] * 5,
        out_shape=[row] * 5,
    )(box_features, boxes_t, w_all, b_all)

    lane = lambda a: a.reshape(_NP // 128, 128)
    out8 = pl.pallas_call(
        _select_body,
        out_shape=jax.ShapeDtypeStruct((8, 128), f32),
        scratch_shapes=[pltpu.VMEM((_NP // 128, 128), f32)] * 3,
    )(lane(score), lane(x0), lane(y0), lane(x1), lane(y1))
    return out8[:5, :_DET].T


# speedup vs baseline: 1.4804x; 1.0443x over previous
import math

import jax
import jax.numpy as jnp
from jax import lax
from jax.experimental import pallas as pl
from jax.experimental.pallas import tpu as pltpu

_N = 5000
_NP = 5120
_FEAT = 1024
_NC = 80
_SCORE_THRESH = 0.05
_NMS_THRESH = 0.5
_DET = 100
_SCALE_CLAMP = math.log(1000.0 / 16.0)
_ROWS = 1024
_GRID = _NP // _ROWS


def _dense_body(x_ref, bxt_ref, w_ref, b_ref, s_ref, x0_ref, y0_ref, x1_ref, y1_ref):
    xb = x_ref[...]
    acc = lax.dot_general(w_ref[...], xb, (((0,), (1,)), ((), ())),
                          preferred_element_type=jnp.float32)
    acc = acc + b_ref[...]
    ci = lax.broadcasted_iota(jnp.int32, acc.shape, 0)
    neg = -jnp.inf
    m_all = jnp.max(jnp.where(ci < _NC + 1, acc, neg), axis=0, keepdims=True)
    m_fg = jnp.max(jnp.where(ci < _NC, acc, neg), axis=0, keepdims=True)
    e = jnp.exp(jnp.where(ci < _NC + 1, acc - m_all, neg))
    s_sum = jnp.sum(e, axis=0, keepdims=True)
    score = jnp.exp(m_fg - m_all) / s_sum

    dx = acc[81:82, :] / 10.0
    dy = acc[82:83, :] / 10.0
    dw = jnp.minimum(acc[83:84, :] / 5.0, _SCALE_CLAMP)
    dh = jnp.minimum(acc[84:85, :] / 5.0, _SCALE_CLAMP)
    px0 = bxt_ref[0:1, :]
    py0 = bxt_ref[1:2, :]
    px1 = bxt_ref[2:3, :]
    py1 = bxt_ref[3:4, :]
    widths = px1 - px0
    heights = py1 - py0
    ctr_x = px0 + 0.5 * widths
    ctr_y = py0 + 0.5 * heights
    pcx = dx * widths + ctr_x
    pcy = dy * heights + ctr_y
    pw = jnp.exp(dw) * widths
    ph = jnp.exp(dh) * heights
    s_ref[...] = score
    x0_ref[...] = pcx - 0.5 * pw
    y0_ref[...] = pcy - 0.5 * ph
    x1_ref[...] = pcx + 0.5 * pw
    y1_ref[...] = pcy + 0.5 * ph


def _allmax(a):
    return jnp.max(jnp.max(a, axis=0, keepdims=True), axis=1, keepdims=True)


def _select_body(s_ref, x0_ref, y0_ref, x1_ref, y1_ref, o_ref,
                 a_ref, b_ref, area_ref):
    sc = s_ref[...]
    bx0 = x0_ref[...]
    by0 = y0_ref[...]
    bx1 = x1_ref[...]
    by1 = y1_ref[...]
    fr = lax.broadcasted_iota(jnp.int32, sc.shape, 0)
    fc = lax.broadcasted_iota(jnp.int32, sc.shape, 1)
    flatf = (fr * 128 + fc).astype(jnp.float32)
    real = (fr * 128 + fc) < _N
    valid = real & (sc > _SCORE_THRESH)
    neg = -jnp.inf
    a_ref[...] = jnp.where(valid, sc, neg)
    b_ref[...] = jnp.where(real, jnp.where(valid, sc, -1.0), neg)
    area_ref[...] = (bx1 - bx0) * (by1 - by0)
    o_ref[...] = jnp.zeros(o_ref.shape, jnp.float32)

    def body(t, carry):
        a = a_ref[...]
        b = b_ref[...]
        m1 = _allmax(a)
        m2 = _allmax(b)
        p1 = m1 > jnp.float32(-1e30)
        pool = jnp.where(p1, a, b)
        m = jnp.where(p1, m1, m2)
        cand = pool == m

        def pickm(msk, arr):
            return jnp.max(jnp.max(jnp.where(msk, arr, neg),
                                   axis=0, keepdims=True), axis=1, keepdims=True)

        j = jnp.min(jnp.min(jnp.where(cand, flatf, jnp.float32(jnp.inf)),
                            axis=0, keepdims=True), axis=1, keepdims=True)
        sel = flatf == j

        jx0 = pickm(sel, bx0)
        jy0 = pickm(sel, by0)
        jx1 = pickm(sel, bx1)
        jy1 = pickm(sel, by1)
        jsc = pickm(sel, sc)
        jar = (jx1 - jx0) * (jy1 - jy0)
        w = jnp.maximum(jnp.minimum(bx1, jx1) - jnp.maximum(bx0, jx0), 0.0)
        h = jnp.maximum(jnp.minimum(by1, jy1) - jnp.maximum(by0, jy0), 0.0)
        inter = w * h
        iou = inter / (jar + area_ref[...] - inter + 1e-9)
        supp = jnp.logical_and(p1, iou > _NMS_THRESH)
        a_ref[...] = jnp.where(supp | sel, neg, a)
        b_ref[...] = jnp.where(sel, neg, b)

        sub8 = lax.broadcasted_iota(jnp.int32, (8, 128), 0)
        lane8 = lax.broadcasted_iota(jnp.int32, (8, 128), 1)
        vals = jnp.where(sub8 == 0, jx0,
               jnp.where(sub8 == 1, jy0,
               jnp.where(sub8 == 2, jx1,
               jnp.where(sub8 == 3, jy1, jsc))))
        o_ref[...] = o_ref[...] + jnp.where(lane8 == t, vals, 0.0)
        return carry

    lax.fori_loop(0, _DET, body, 0)


def kernel(box_features, proposal_boxes, W_cls, b_cls, W_box, b_box):
    f32 = jnp.float32
    w_all = jnp.zeros((_FEAT, 128), f32)
    w_all = w_all.at[:, : _NC + 1].set(W_cls).at[:, _NC + 1 : _NC + 5].set(W_box)
    b_all = jnp.zeros((128, 1), f32)
    b_all = b_all.at[: _NC + 1, 0].set(b_cls).at[_NC + 1 : _NC + 5, 0].set(b_box)
    boxes_t = proposal_boxes.T

    row = jax.ShapeDtypeStruct((1, _NP), f32)
    score, x0, y0, x1, y1 = pl.pallas_call(
        _dense_body,
        grid=(_GRID,),
        in_specs=[
            pl.BlockSpec((_ROWS, _FEAT), lambda i: (i, 0)),
            pl.BlockSpec((4, _ROWS), lambda i: (0, i)),
            pl.BlockSpec((_FEAT, 128), lambda i: (0, 0)),
            pl.BlockSpec((128, 1), lambda i: (0, 0)),
        ],
        out_specs=[pl.BlockSpec((1, _ROWS), lambda i: (0, i))] * 5,
        out_shape=[row] * 5,
    )(box_features, boxes_t, w_all, b_all)

    lane = lambda a: a.reshape(_NP // 128, 128)
    out8 = pl.pallas_call(
        _select_body,
        out_shape=jax.ShapeDtypeStruct((8, 128), f32),
        scratch_shapes=[pltpu.VMEM((_NP // 128, 128), f32)] * 3,
    )(lane(score), lane(x0), lane(y0), lane(x1), lane(y1))
    return out8[:5, :_DET].T


# speedup vs baseline: 1.5013x; 1.0141x over previous
import math

import jax
import jax.numpy as jnp
from jax import lax
from jax.experimental import pallas as pl
from jax.experimental.pallas import tpu as pltpu

_N = 5000
_NP = 5120
_FEAT = 1024
_NC = 80
_SCORE_THRESH = 0.05
_NMS_THRESH = 0.5
_DET = 100
_SCALE_CLAMP = math.log(1000.0 / 16.0)
_ROWS = 2560
_GRID = _NP // _ROWS


def _dense_body(x_ref, bxt_ref, w_ref, b_ref, s_ref, x0_ref, y0_ref, x1_ref, y1_ref):
    xb = x_ref[...]
    acc = lax.dot_general(w_ref[...], xb, (((0,), (1,)), ((), ())),
                          preferred_element_type=jnp.float32)
    acc = acc + b_ref[...]
    ci = lax.broadcasted_iota(jnp.int32, acc.shape, 0)
    neg = -jnp.inf
    m_all = jnp.max(jnp.where(ci < _NC + 1, acc, neg), axis=0, keepdims=True)
    m_fg = jnp.max(jnp.where(ci < _NC, acc, neg), axis=0, keepdims=True)
    e = jnp.exp(jnp.where(ci < _NC + 1, acc - m_all, neg))
    s_sum = jnp.sum(e, axis=0, keepdims=True)
    score = jnp.exp(m_fg - m_all) / s_sum

    dx = acc[81:82, :] / 10.0
    dy = acc[82:83, :] / 10.0
    dw = jnp.minimum(acc[83:84, :] / 5.0, _SCALE_CLAMP)
    dh = jnp.minimum(acc[84:85, :] / 5.0, _SCALE_CLAMP)
    px0 = bxt_ref[0:1, :]
    py0 = bxt_ref[1:2, :]
    px1 = bxt_ref[2:3, :]
    py1 = bxt_ref[3:4, :]
    widths = px1 - px0
    heights = py1 - py0
    ctr_x = px0 + 0.5 * widths
    ctr_y = py0 + 0.5 * heights
    pcx = dx * widths + ctr_x
    pcy = dy * heights + ctr_y
    pw = jnp.exp(dw) * widths
    ph = jnp.exp(dh) * heights
    s_ref[...] = score
    x0_ref[...] = pcx - 0.5 * pw
    y0_ref[...] = pcy - 0.5 * ph
    x1_ref[...] = pcx + 0.5 * pw
    y1_ref[...] = pcy + 0.5 * ph


def _allmax(a):
    return jnp.max(jnp.max(a, axis=0, keepdims=True), axis=1, keepdims=True)


def _select_body(s_ref, x0_ref, y0_ref, x1_ref, y1_ref, o_ref,
                 a_ref, b_ref, area_ref):
    sc = s_ref[...]
    bx0 = x0_ref[...]
    by0 = y0_ref[...]
    bx1 = x1_ref[...]
    by1 = y1_ref[...]
    fr = lax.broadcasted_iota(jnp.int32, sc.shape, 0)
    fc = lax.broadcasted_iota(jnp.int32, sc.shape, 1)
    flatf = (fr * 128 + fc).astype(jnp.float32)
    real = (fr * 128 + fc) < _N
    valid = real & (sc > _SCORE_THRESH)
    neg = -jnp.inf
    a_ref[...] = jnp.where(valid, sc, neg)
    b_ref[...] = jnp.where(real, jnp.where(valid, sc, -1.0), neg)
    area_ref[...] = (bx1 - bx0) * (by1 - by0)
    o_ref[...] = jnp.zeros(o_ref.shape, jnp.float32)

    def body(t, carry):
        a = a_ref[...]
        b = b_ref[...]
        m1 = _allmax(a)
        m2 = _allmax(b)
        p1 = m1 > jnp.float32(-1e30)
        pool = jnp.where(p1, a, b)
        m = jnp.where(p1, m1, m2)
        cand = pool == m

        def pickm(msk, arr):
            return jnp.max(jnp.max(jnp.where(msk, arr, neg),
                                   axis=0, keepdims=True), axis=1, keepdims=True)

        j = jnp.min(jnp.min(jnp.where(cand, flatf, jnp.float32(jnp.inf)),
                            axis=0, keepdims=True), axis=1, keepdims=True)
        sel = flatf == j

        jx0 = pickm(sel, bx0)
        jy0 = pickm(sel, by0)
        jx1 = pickm(sel, bx1)
        jy1 = pickm(sel, by1)
        jsc = pickm(sel, sc)
        jar = (jx1 - jx0) * (jy1 - jy0)
        w = jnp.maximum(jnp.minimum(bx1, jx1) - jnp.maximum(bx0, jx0), 0.0)
        h = jnp.maximum(jnp.minimum(by1, jy1) - jnp.maximum(by0, jy0), 0.0)
        inter = w * h
        iou = inter / (jar + area_ref[...] - inter + 1e-9)
        supp = jnp.logical_and(p1, iou > _NMS_THRESH)
        a_ref[...] = jnp.where(supp | sel, neg, a)
        b_ref[...] = jnp.where(sel, neg, b)

        sub8 = lax.broadcasted_iota(jnp.int32, (8, 128), 0)
        lane8 = lax.broadcasted_iota(jnp.int32, (8, 128), 1)
        vals = jnp.where(sub8 == 0, jx0,
               jnp.where(sub8 == 1, jy0,
               jnp.where(sub8 == 2, jx1,
               jnp.where(sub8 == 3, jy1, jsc))))
        o_ref[...] = o_ref[...] + jnp.where(lane8 == t, vals, 0.0)
        return carry

    lax.fori_loop(0, _DET, body, 0)


def kernel(box_features, proposal_boxes, W_cls, b_cls, W_box, b_box):
    f32 = jnp.float32
    w_all = jnp.zeros((_FEAT, 128), f32)
    w_all = w_all.at[:, : _NC + 1].set(W_cls).at[:, _NC + 1 : _NC + 5].set(W_box)
    b_all = jnp.zeros((128, 1), f32)
    b_all = b_all.at[: _NC + 1, 0].set(b_cls).at[_NC + 1 : _NC + 5, 0].set(b_box)
    boxes_t = proposal_boxes.T

    row = jax.ShapeDtypeStruct((1, _NP), f32)
    score, x0, y0, x1, y1 = pl.pallas_call(
        _dense_body,
        grid=(_GRID,),
        in_specs=[
            pl.BlockSpec((_ROWS, _FEAT), lambda i: (i, 0)),
            pl.BlockSpec((4, _ROWS), lambda i: (0, i)),
            pl.BlockSpec((_FEAT, 128), lambda i: (0, 0)),
            pl.BlockSpec((128, 1), lambda i: (0, 0)),
        ],
        out_specs=[pl.BlockSpec((1, _ROWS), lambda i: (0, i))] * 5,
        out_shape=[row] * 5,
    )(box_features, boxes_t, w_all, b_all)

    lane = lambda a: a.reshape(_NP // 128, 128)
    out8 = pl.pallas_call(
        _select_body,
        out_shape=jax.ShapeDtypeStruct((8, 128), f32),
        scratch_shapes=[pltpu.VMEM((_NP // 128, 128), f32)] * 3,
    )(lane(score), lane(x0), lane(y0), lane(x1), lane(y1))
    return out8[:5, :_DET].T
